# baseline (speedup 1.0000x reference)
import functools

import jax
import jax.numpy as jnp
from jax import lax
from jax.experimental import pallas as pl
from jax.experimental.pallas import tpu as pltpu
from jax.experimental.pallas import tpu_sc as plsc

VOCAB = 1000000
EMBED_DIM = 32
BATCH = 16384
FIELDS = 26
B_TOTAL = BATCH * FIELDS

PACK = 128 // EMBED_DIM
VROWS = VOCAB // PACK
TBLK = 1024
TGRID = -(-VROWS // TBLK)

NC = 2
NS = 16
NW = NC * NS
B_PER_W = B_TOTAL // NW
CHUNK = 1664
N_CHUNKS = B_PER_W // CHUNK


def _retile_body(t_ref, out_ref):
    x = t_ref[...].reshape(EMBED_DIM, TBLK, PACK)
    out_ref[...] = x.transpose((1, 2, 0)).reshape(TBLK, PACK * EMBED_DIM)


def _retile_table(table_t):
    return pl.pallas_call(
        _retile_body,
        grid=(TGRID,),
        in_specs=[pl.BlockSpec((EMBED_DIM, PACK * TBLK), lambda i: (0, i))],
        out_specs=pl.BlockSpec((TBLK, PACK * EMBED_DIM), lambda i: (i, 0)),
        out_shape=jax.ShapeDtypeStruct((VROWS, PACK * EMBED_DIM), jnp.float32),
    )(table_t)


@functools.partial(
    pl.kernel,
    mesh=plsc.VectorSubcoreMesh(core_axis_name="c", subcore_axis_name="s"),
    out_type=jax.ShapeDtypeStruct((B_TOTAL, EMBED_DIM), jnp.float32),
    scratch_types=[
        pltpu.VMEM((CHUNK,), jnp.int32),
        pltpu.VMEM((CHUNK, EMBED_DIM), jnp.float32),
        pltpu.SemaphoreType.DMA,
    ],
    compiler_params=pltpu.CompilerParams(use_tc_tiling_on_sc=False),
)
def _embed_gather(idx_hbm, table_hbm, out_hbm, idx_v, rows_v, sem):
    wid = lax.axis_index("s") * NC + lax.axis_index("c")
    base = wid * B_PER_W

    def body(i, carry):
        off = base + i * CHUNK
        pltpu.sync_copy(idx_hbm.at[pl.ds(off, CHUNK)], idx_v)
        pltpu.async_copy(table_hbm.at[idx_v], rows_v, sem).wait()
        pltpu.sync_copy(rows_v, out_hbm.at[pl.ds(off, CHUNK)])
        return carry

    lax.fori_loop(0, N_CHUNKS, body, 0)


def kernel(indices, table):
    table_rm = _retile_table(table.T).reshape(VOCAB, EMBED_DIM)
    flat_idx = indices.reshape(-1).astype(jnp.int32)
    out = _embed_gather(flat_idx, table_rm)
    return out.reshape(BATCH, FIELDS, EMBED_DIM)


# speedup vs baseline: 1.6271x; 1.6271x over previous
import functools

import jax
import jax.numpy as jnp
from jax import lax
from jax.experimental import pallas as pl
from jax.experimental.pallas import tpu as pltpu
from jax.experimental.pallas import tpu_sc as plsc

VOCAB = 1000000
EMBED_DIM = 32
BATCH = 16384
FIELDS = 26
B_TOTAL = BATCH * FIELDS

NC = 2
NS = 16
NW = NC * NS

LANES = 128
PACK = LANES // EMBED_DIM
RT_TILES = -(-VOCAB // LANES)
CT_TILES = EMBED_DIM // 8
DST_ROWS = 32 * RT_TILES
VOCAB_PAD = DST_ROWS * PACK
RT_PER_W = -(-RT_TILES // NW)
TAIL_COLS = VOCAB - (RT_TILES - 1) * LANES

B_PER_W = B_TOTAL // NW
CHUNK = 1664
N_CHUNKS = B_PER_W // CHUNK

_MESH = dict(core_axis_name="c", subcore_axis_name="s")


@functools.partial(
    pl.kernel,
    mesh=plsc.VectorSubcoreMesh(**_MESH),
    out_type=jax.ShapeDtypeStruct((DST_ROWS, LANES), jnp.float32),
    scratch_types=[
        pltpu.VMEM((CT_TILES, 8, LANES), jnp.float32),
        pltpu.VMEM((32, LANES), jnp.float32),
    ],
    compiler_params=pltpu.CompilerParams(needs_layout_passes=False),
)
def _retile_sc(t_hbm, out_hbm, src_v, dst_v):
    wid = lax.axis_index("s") * NC + lax.axis_index("c")
    iota = lax.iota(jnp.int32, 16)
    ict = [(16 * h + iota) >> 3 for h in (0, 1)]
    icin = [(16 * h + iota) & 7 for h in (0, 1)]

    def body(i, carry):
        rt = wid * RT_PER_W + i

        @pl.when(rt < RT_TILES - 1)
        def _full_in():
            for ct in range(CT_TILES):
                pltpu.sync_copy(
                    t_hbm.at[pl.ds(8 * ct, 8), pl.ds(rt * LANES, LANES)],
                    src_v.at[ct],
                )

        @pl.when(rt == RT_TILES - 1)
        def _tail_in():
            for ct in range(CT_TILES):
                pltpu.sync_copy(
                    t_hbm.at[pl.ds(8 * ct, 8), pl.ds(rt * LANES, TAIL_COLS)],
                    src_v.at[ct, :, pl.ds(0, TAIL_COLS)],
                )

        @pl.when(rt < RT_TILES)
        def _repack_out():
            for p in range(32):
                for k in range(8):
                    irin = jnp.full((16,), 4 * p + (k >> 1), jnp.int32)
                    vals = plsc.load_gather(src_v, [ict[k & 1], icin[k & 1], irin])
                    dst_v[p, pl.ds(16 * k, 16)] = vals
            pltpu.sync_copy(dst_v, out_hbm.at[pl.ds(rt * 32, 32)])

        return carry

    lax.fori_loop(0, RT_PER_W, body, 0)


@functools.partial(
    pl.kernel,
    mesh=plsc.VectorSubcoreMesh(**_MESH),
    out_type=jax.ShapeDtypeStruct((B_TOTAL, EMBED_DIM), jnp.float32),
    scratch_types=[
        pltpu.VMEM((CHUNK,), jnp.int32),
        pltpu.VMEM((CHUNK, EMBED_DIM), jnp.float32),
        pltpu.SemaphoreType.DMA,
    ],
    compiler_params=pltpu.CompilerParams(use_tc_tiling_on_sc=False),
)
def _embed_gather(idx_hbm, table_hbm, out_hbm, idx_v, rows_v, sem):
    wid = lax.axis_index("s") * NC + lax.axis_index("c")
    base = wid * B_PER_W

    def body(i, carry):
        off = base + i * CHUNK
        pltpu.sync_copy(idx_hbm.at[pl.ds(off, CHUNK)], idx_v)
        pltpu.async_copy(table_hbm.at[idx_v], rows_v, sem).wait()
        pltpu.sync_copy(rows_v, out_hbm.at[pl.ds(off, CHUNK)])
        return carry

    lax.fori_loop(0, N_CHUNKS, body, 0)


def kernel(indices, table):
    packed = _retile_sc(table.T)
    table_rm = packed.reshape(VOCAB_PAD, EMBED_DIM)
    flat_idx = indices.reshape(-1).astype(jnp.int32)
    out = _embed_gather(flat_idx, table_rm)
    return out.reshape(BATCH, FIELDS, EMBED_DIM)
